# baseline (speedup 1.0000x reference)
import functools

import jax
import jax.numpy as jnp
from jax import lax
from jax.experimental import pallas as pl
from jax.experimental.pallas import tpu as pltpu
from jax.experimental.pallas import tpu_sc as plsc

_NC = 2
_NS = 16
_NW = _NC * _NS
_L = 16
_D = 64
_CH = 128


def _sc_body(users, items, dist_tab, rp_tab, ug_tab, ud_tab, up_tab, un_tab,
             ibp_tab, ibn_tab, uep_tab, iep_tab, uen_tab, ien_tab,
             pos_out, neg_out, gam_out, dlt_out, rpo_out, dst_out,
             u_idx, i_idx, rp_v, g_v, d_v, up_v, un_v, ibp_v, ibn_v,
             dist_v, pos_v, neg_v, uep_v, iep_v, uen_v, ien_v,
             sem_small, sem_a, sem_b):
    bpw = pos_v.shape[0]
    nch = bpw // _CH
    wid = lax.axis_index("s") * _NC + lax.axis_index("c")
    base = wid * bpw

    for c in range(nch):
        pltpu.sync_copy(users.at[pl.ds(base + c * _CH, _CH)], u_idx.at[c])
        pltpu.sync_copy(items.at[pl.ds(base + c * _CH, _CH)], i_idx.at[c])

    small = []
    for c in range(nch):
        sl = pl.ds(c * _CH, _CH)
        uc = u_idx.at[c]
        ic = i_idx.at[c]
        small.append(pltpu.async_copy(rp_tab.at[uc], rp_v.at[sl], sem_small))
        small.append(pltpu.async_copy(ug_tab.at[uc], g_v.at[sl], sem_small))
        small.append(pltpu.async_copy(ud_tab.at[uc], d_v.at[sl], sem_small))
        small.append(pltpu.async_copy(up_tab.at[uc], up_v.at[sl], sem_small))
        small.append(pltpu.async_copy(un_tab.at[uc], un_v.at[sl], sem_small))
        small.append(pltpu.async_copy(ibp_tab.at[ic], ibp_v.at[sl], sem_small))
        small.append(pltpu.async_copy(ibn_tab.at[ic], ibn_v.at[sl], sem_small))
        small.append(pltpu.async_copy(dist_tab.at[ic], dist_v.at[sl], sem_small))

    def fire_big(c):
        s = c % 2
        sem = sem_a if s == 0 else sem_b
        uc = u_idx.at[c]
        ic = i_idx.at[c]
        return [pltpu.async_copy(uep_tab.at[uc], uep_v.at[s], sem),
                pltpu.async_copy(iep_tab.at[ic], iep_v.at[s], sem),
                pltpu.async_copy(uen_tab.at[uc], uen_v.at[s], sem),
                pltpu.async_copy(ien_tab.at[ic], ien_v.at[s], sem)]

    pend = fire_big(0)
    for dsc in small:
        dsc.wait()

    for c in range(nch):
        for dsc in pend:
            dsc.wait()
        if c + 1 < nch:
            pend = fire_big(c + 1)
        s = c % 2
        uev, iev, unv, inv = uep_v.at[s], iep_v.at[s], uen_v.at[s], ien_v.at[s]
        off = c * _CH

        def group(g, _):
            rows = g * _L + lax.iota(jnp.int32, _L)
            zero = jnp.zeros((_L,), jnp.float32)
            accp = [zero, zero, zero, zero]
            accn = [zero, zero, zero, zero]
            for d in range(_D):
                cols = jnp.full((_L,), d, jnp.int32)
                accp[d % 4] = accp[d % 4] + (plsc.load_gather(uev, [rows, cols])
                                             * plsc.load_gather(iev, [rows, cols]))
                accn[d % 4] = accn[d % 4] + (plsc.load_gather(unv, [rows, cols])
                                             * plsc.load_gather(inv, [rows, cols]))
            dot_p = (accp[0] + accp[1]) + (accp[2] + accp[3])
            dot_n = (accn[0] + accn[1]) + (accn[2] + accn[3])
            sl16 = pl.ds(off + g * _L, _L)
            pos_v[sl16] = dot_p + up_v[sl16] + ibp_v[sl16]
            neg_v[sl16] = dot_n + un_v[sl16] + ibn_v[sl16]
            return 0

        lax.fori_loop(0, _CH // _L, group, 0)

    pltpu.sync_copy(pos_v, pos_out.at[pl.ds(base, bpw)])
    pltpu.sync_copy(neg_v, neg_out.at[pl.ds(base, bpw)])
    pltpu.sync_copy(g_v, gam_out.at[pl.ds(base, bpw)])
    pltpu.sync_copy(d_v, dlt_out.at[pl.ds(base, bpw)])
    pltpu.sync_copy(rp_v, rpo_out.at[pl.ds(base, bpw)])
    pltpu.sync_copy(dist_v, dst_out.at[pl.ds(base, bpw)])


def _tc_body(gbg, gbd, gbp, gbn, pos, neg, gam, dlt, rp, dist, out):
    gamma = gbg[0, 0] + gam[...]
    delta = gbd[0, 0] + dlt[...]
    pv = gbp[0, 0] + pos[...]
    nv = gbn[0, 0] + neg[...]
    r = rp[...]
    acc = jnp.zeros_like(r)
    for k in range(5):
        t = jnp.tanh((k + 1.0) - r)
        val = jnp.where(t > 0.0, pv * t, nv * t)
        dk = dist[k]
        num = delta * jnp.exp(gamma * jnp.log(dk))
        den = num + jnp.exp(gamma * jnp.log(1.0 - dk))
        acc = acc + (num / den) * val
    out[...] = acc


def kernel(users, items, distribution, item_price, ref_point, gb_g, ub_g,
           gb_d, ub_d, gb_p, ub_p, ib_p, ue_p, ie_p, gb_n, ub_n, ib_n,
           ue_n, ie_n):
    del item_price
    B = users.shape[0]
    bpw = B // _NW
    nch = bpw // _CH
    f32 = jnp.float32
    u = users.astype(jnp.int32)
    it = items.astype(jnp.int32)

    mesh = plsc.VectorSubcoreMesh(core_axis_name="c", subcore_axis_name="s")
    vecs = jax.ShapeDtypeStruct((B,), f32)
    sc = pl.kernel(
        _sc_body,
        out_type=[vecs, vecs, vecs, vecs, vecs,
                  jax.ShapeDtypeStruct((B, 5), f32)],
        mesh=mesh,
        compiler_params=pltpu.CompilerParams(needs_layout_passes=False,
                                             use_tc_tiling_on_sc=False),
        scratch_types=[
            pltpu.VMEM((nch, _CH), jnp.int32),
            pltpu.VMEM((nch, _CH), jnp.int32),
            pltpu.VMEM((bpw,), f32),
            pltpu.VMEM((bpw,), f32),
            pltpu.VMEM((bpw,), f32),
            pltpu.VMEM((bpw,), f32),
            pltpu.VMEM((bpw,), f32),
            pltpu.VMEM((bpw,), f32),
            pltpu.VMEM((bpw,), f32),
            pltpu.VMEM((bpw, 5), f32),
            pltpu.VMEM((bpw,), f32),
            pltpu.VMEM((bpw,), f32),
            pltpu.VMEM((2, _CH, _D), f32),
            pltpu.VMEM((2, _CH, _D), f32),
            pltpu.VMEM((2, _CH, _D), f32),
            pltpu.VMEM((2, _CH, _D), f32),
            pltpu.SemaphoreType.DMA,
            pltpu.SemaphoreType.DMA,
            pltpu.SemaphoreType.DMA,
        ],
    )
    pos, neg, gam, dlt, rpo, dst = sc(
        u, it, distribution, ref_point.reshape(-1), ub_g.reshape(-1),
        ub_d.reshape(-1), ub_p.reshape(-1), ub_n.reshape(-1),
        ib_p.reshape(-1), ib_n.reshape(-1), ue_p, ie_p, ue_n, ie_n)

    M = B // 128
    dist_t = dst.T.reshape(5, M, 128)
    smem = pl.BlockSpec(memory_space=pltpu.SMEM)
    vmem = pl.BlockSpec(memory_space=pltpu.VMEM)
    out2d = pl.pallas_call(
        _tc_body,
        out_shape=jax.ShapeDtypeStruct((M, 128), f32),
        in_specs=[smem, smem, smem, smem, vmem, vmem, vmem, vmem, vmem, vmem],
        out_specs=vmem,
    )(gb_g, gb_d, gb_p, gb_n, pos.reshape(M, 128), neg.reshape(M, 128),
      gam.reshape(M, 128), dlt.reshape(M, 128), rpo.reshape(M, 128), dist_t)
    return out2d.reshape(B)


# speedup vs baseline: 1.4160x; 1.4160x over previous
import jax
import jax.numpy as jnp
from jax import lax
from jax.experimental import pallas as pl
from jax.experimental.pallas import tpu as pltpu
from jax.experimental.pallas import tpu_sc as plsc

_NC = 2
_NS = 16
_NW = _NC * _NS
_L = 16
_D = 64
_CH = 128
_BS = 512


def _prep_body(uept, uent, iept, ient, distt, ue2, ie2, d0, d1, d2, d3, d4):
    ue2[:, 0:_D] = uept[...].T
    ue2[:, _D:2 * _D] = uent[...].T
    ie2[:, 0:_D] = iept[...].T
    ie2[:, _D:2 * _D] = ient[...].T
    dd = distt[...]
    d0[...] = dd[0]
    d1[...] = dd[1]
    d2[...] = dd[2]
    d3[...] = dd[3]
    d4[...] = dd[4]


def _sc_body(users, items, ue2, ie2, rp_tab, ug_tab, ud_tab, up_tab, un_tab,
             ibp_tab, ibn_tab, d0_tab, d1_tab, d2_tab, d3_tab, d4_tab,
             pos_out, neg_out, gam_out, dlt_out, rpo_out,
             w0_out, w1_out, w2_out, w3_out, w4_out,
             u_idx, i_idx, rp_v, g_v, d_v, up_v, un_v, ibp_v, ibn_v,
             dv0, dv1, dv2, dv3, dv4, pos_v, neg_v, ue_b, ie_b,
             sem_small, sem_a, sem_b):
    bpw = pos_v.shape[0]
    nch = bpw // _CH
    wid = lax.axis_index("s") * _NC + lax.axis_index("c")
    base = wid * bpw

    pltpu.sync_copy(users.at[pl.ds(base, bpw)], u_idx)
    pltpu.sync_copy(items.at[pl.ds(base, bpw)], i_idx)

    small = [
        pltpu.async_copy(rp_tab.at[u_idx], rp_v, sem_small),
        pltpu.async_copy(ug_tab.at[u_idx], g_v, sem_small),
        pltpu.async_copy(ud_tab.at[u_idx], d_v, sem_small),
        pltpu.async_copy(up_tab.at[u_idx], up_v, sem_small),
        pltpu.async_copy(un_tab.at[u_idx], un_v, sem_small),
        pltpu.async_copy(ibp_tab.at[i_idx], ibp_v, sem_small),
        pltpu.async_copy(ibn_tab.at[i_idx], ibn_v, sem_small),
        pltpu.async_copy(d0_tab.at[i_idx], dv0, sem_small),
        pltpu.async_copy(d1_tab.at[i_idx], dv1, sem_small),
        pltpu.async_copy(d2_tab.at[i_idx], dv2, sem_small),
        pltpu.async_copy(d3_tab.at[i_idx], dv3, sem_small),
        pltpu.async_copy(d4_tab.at[i_idx], dv4, sem_small),
    ]

    def fire_big(c):
        s = c % 2
        sem = sem_a if s == 0 else sem_b
        sl = pl.ds(c * _CH, _CH)
        return [pltpu.async_copy(ue2.at[u_idx.at[sl]], ue_b.at[s], sem),
                pltpu.async_copy(ie2.at[i_idx.at[sl]], ie_b.at[s], sem)]

    pend = fire_big(0)
    for dsc in small:
        dsc.wait()

    for c in range(nch):
        for dsc in pend:
            dsc.wait()
        if c + 1 < nch:
            pend = fire_big(c + 1)
        s = c % 2
        uev, iev = ue_b.at[s], ie_b.at[s]
        off = c * _CH

        def group(g, _):
            rows = g * _L + lax.iota(jnp.int32, _L)
            zero = jnp.zeros((_L,), jnp.float32)
            accp = [zero, zero, zero, zero]
            accn = [zero, zero, zero, zero]
            for d in range(_D):
                cols = jnp.full((_L,), d, jnp.int32)
                accp[d % 4] = accp[d % 4] + (plsc.load_gather(uev, [rows, cols])
                                             * plsc.load_gather(iev, [rows, cols]))
                colsn = jnp.full((_L,), _D + d, jnp.int32)
                accn[d % 4] = accn[d % 4] + (plsc.load_gather(uev, [rows, colsn])
                                             * plsc.load_gather(iev, [rows, colsn]))
            dot_p = (accp[0] + accp[1]) + (accp[2] + accp[3])
            dot_n = (accn[0] + accn[1]) + (accn[2] + accn[3])
            sl16 = pl.ds(off + g * _L, _L)
            pos_v[sl16] = dot_p + up_v[sl16] + ibp_v[sl16]
            neg_v[sl16] = dot_n + un_v[sl16] + ibn_v[sl16]
            return 0

        lax.fori_loop(0, _CH // _L, group, 0)

    osl = pl.ds(base, bpw)
    pltpu.sync_copy(pos_v, pos_out.at[osl])
    pltpu.sync_copy(neg_v, neg_out.at[osl])
    pltpu.sync_copy(g_v, gam_out.at[osl])
    pltpu.sync_copy(d_v, dlt_out.at[osl])
    pltpu.sync_copy(rp_v, rpo_out.at[osl])
    pltpu.sync_copy(dv0, w0_out.at[osl])
    pltpu.sync_copy(dv1, w1_out.at[osl])
    pltpu.sync_copy(dv2, w2_out.at[osl])
    pltpu.sync_copy(dv3, w3_out.at[osl])
    pltpu.sync_copy(dv4, w4_out.at[osl])


def _tc_body(gbg, gbd, gbp, gbn, pos, neg, gam, dlt, rp,
             q0, q1, q2, q3, q4, out):
    gamma = gbg[0, 0] + gam[...]
    delta = gbd[0, 0] + dlt[...]
    pv = gbp[0, 0] + pos[...]
    nv = gbn[0, 0] + neg[...]
    r = rp[...]
    acc = jnp.zeros_like(r)
    for k, q in enumerate((q0, q1, q2, q3, q4)):
        t = jnp.tanh((k + 1.0) - r)
        val = jnp.where(t > 0.0, pv * t, nv * t)
        dk = q[...]
        num = delta * jnp.exp(gamma * jnp.log(dk))
        den = num + jnp.exp(gamma * jnp.log(1.0 - dk))
        acc = acc + (num / den) * val
    out[...] = acc


def kernel(users, items, distribution, item_price, ref_point, gb_g, ub_g,
           gb_d, ub_d, gb_p, ub_p, ib_p, ue_p, ie_p, gb_n, ub_n, ib_n,
           ue_n, ie_n):
    del item_price
    B = users.shape[0]
    bpw = B // _NW
    NU = ue_p.shape[0]
    NI = ie_p.shape[0]
    f32 = jnp.float32
    u = users.astype(jnp.int32)
    it = items.astype(jnp.int32)

    grid = (NU + _BS - 1) // _BS
    tvec = jax.ShapeDtypeStruct((NI,), f32)
    ue2, ie2, d0, d1, d2, d3, d4 = pl.pallas_call(
        _prep_body,
        grid=(grid,),
        in_specs=[
            pl.BlockSpec((_D, _BS), lambda j: (0, j)),
            pl.BlockSpec((_D, _BS), lambda j: (0, j)),
            pl.BlockSpec((_D, _BS), lambda j: (0, j)),
            pl.BlockSpec((_D, _BS), lambda j: (0, j)),
            pl.BlockSpec((5, _BS), lambda j: (0, j)),
        ],
        out_specs=[
            pl.BlockSpec((_BS, 2 * _D), lambda j: (j, 0)),
            pl.BlockSpec((_BS, 2 * _D), lambda j: (j, 0)),
            pl.BlockSpec((_BS,), lambda j: (j,)),
            pl.BlockSpec((_BS,), lambda j: (j,)),
            pl.BlockSpec((_BS,), lambda j: (j,)),
            pl.BlockSpec((_BS,), lambda j: (j,)),
            pl.BlockSpec((_BS,), lambda j: (j,)),
        ],
        out_shape=[
            jax.ShapeDtypeStruct((NU, 2 * _D), f32),
            jax.ShapeDtypeStruct((NI, 2 * _D), f32),
            tvec, tvec, tvec, tvec, tvec,
        ],
    )(ue_p.T, ue_n.T, ie_p.T, ie_n.T, distribution.T)

    mesh = plsc.VectorSubcoreMesh(core_axis_name="c", subcore_axis_name="s")
    vecs = jax.ShapeDtypeStruct((B,), f32)
    sc = pl.kernel(
        _sc_body,
        out_type=[vecs] * 10,
        mesh=mesh,
        compiler_params=pltpu.CompilerParams(needs_layout_passes=False,
                                             use_tc_tiling_on_sc=True),
        scratch_types=[
            pltpu.VMEM((bpw,), jnp.int32),
            pltpu.VMEM((bpw,), jnp.int32),
            pltpu.VMEM((bpw,), f32),
            pltpu.VMEM((bpw,), f32),
            pltpu.VMEM((bpw,), f32),
            pltpu.VMEM((bpw,), f32),
            pltpu.VMEM((bpw,), f32),
            pltpu.VMEM((bpw,), f32),
            pltpu.VMEM((bpw,), f32),
            pltpu.VMEM((bpw,), f32),
            pltpu.VMEM((bpw,), f32),
            pltpu.VMEM((bpw,), f32),
            pltpu.VMEM((bpw,), f32),
            pltpu.VMEM((bpw,), f32),
            pltpu.VMEM((bpw,), f32),
            pltpu.VMEM((bpw,), f32),
            pltpu.VMEM((2, _CH, 2 * _D), f32),
            pltpu.VMEM((2, _CH, 2 * _D), f32),
            pltpu.SemaphoreType.DMA,
            pltpu.SemaphoreType.DMA,
            pltpu.SemaphoreType.DMA,
        ],
    )
    pos, neg, gam, dlt, rpo, w0, w1, w2, w3, w4 = sc(
        u, it, ue2, ie2, ref_point.reshape(-1), ub_g.reshape(-1),
        ub_d.reshape(-1), ub_p.reshape(-1), ub_n.reshape(-1),
        ib_p.reshape(-1), ib_n.reshape(-1), d0, d1, d2, d3, d4)

    M = B // 128
    r2 = lambda x: x.reshape(M, 128)
    smem = pl.BlockSpec(memory_space=pltpu.SMEM)
    vmem = pl.BlockSpec(memory_space=pltpu.VMEM)
    out2d = pl.pallas_call(
        _tc_body,
        out_shape=jax.ShapeDtypeStruct((M, 128), f32),
        in_specs=[smem] * 4 + [vmem] * 10,
        out_specs=vmem,
    )(gb_g, gb_d, gb_p, gb_n, r2(pos), r2(neg), r2(gam), r2(dlt), r2(rpo),
      r2(w0), r2(w1), r2(w2), r2(w3), r2(w4))
    return out2d.reshape(B)


# speedup vs baseline: 1.8900x; 1.3347x over previous
import jax
import jax.numpy as jnp
from jax import lax
from jax.experimental import pallas as pl
from jax.experimental.pallas import tpu as pltpu
from jax.experimental.pallas import tpu_sc as plsc

_NC = 2
_NS = 16
_NW = _NC * _NS
_L = 16
_D = 64
_CH = 128
_BS = 2048


def _prep_body(uept, uent, iept, ient, distt, ue2, ie2, d0, d1, d2, d3, d4):
    ue2[:, 0:_D] = uept[...].T
    ue2[:, _D:2 * _D] = uent[...].T
    ie2[:, 0:_D] = iept[...].T
    ie2[:, _D:2 * _D] = ient[...].T
    dd = distt[...]
    d0[...] = dd[0]
    d1[...] = dd[1]
    d2[...] = dd[2]
    d3[...] = dd[3]
    d4[...] = dd[4]


def _sc_body(users, items, ue2, ie2, rp_tab, ug_tab, ud_tab, up_tab, un_tab,
             ibp_tab, ibn_tab, d0_tab, d1_tab, d2_tab, d3_tab, d4_tab,
             pos_out, neg_out, gam_out, dlt_out, rpo_out,
             w0_out, w1_out, w2_out, w3_out, w4_out,
             u_idx, i_idx, rp_v, g_v, d_v, up_v, un_v, ibp_v, ibn_v,
             dv0, dv1, dv2, dv3, dv4, pos_v, neg_v, ue_b, ie_b,
             sem_small, sem_a, sem_b):
    bpw = pos_v.shape[0]
    nch = bpw // _CH
    wid = lax.axis_index("s") * _NC + lax.axis_index("c")
    base = wid * bpw

    pltpu.sync_copy(users.at[pl.ds(base, bpw)], u_idx)
    pltpu.sync_copy(items.at[pl.ds(base, bpw)], i_idx)

    small = [
        pltpu.async_copy(rp_tab.at[u_idx], rp_v, sem_small),
        pltpu.async_copy(ug_tab.at[u_idx], g_v, sem_small),
        pltpu.async_copy(ud_tab.at[u_idx], d_v, sem_small),
        pltpu.async_copy(up_tab.at[u_idx], up_v, sem_small),
        pltpu.async_copy(un_tab.at[u_idx], un_v, sem_small),
        pltpu.async_copy(ibp_tab.at[i_idx], ibp_v, sem_small),
        pltpu.async_copy(ibn_tab.at[i_idx], ibn_v, sem_small),
        pltpu.async_copy(d0_tab.at[i_idx], dv0, sem_small),
        pltpu.async_copy(d1_tab.at[i_idx], dv1, sem_small),
        pltpu.async_copy(d2_tab.at[i_idx], dv2, sem_small),
        pltpu.async_copy(d3_tab.at[i_idx], dv3, sem_small),
        pltpu.async_copy(d4_tab.at[i_idx], dv4, sem_small),
    ]

    def fire_big(c):
        s = c % 2
        sem = sem_a if s == 0 else sem_b
        sl = pl.ds(c * _CH, _CH)
        return [pltpu.async_copy(ue2.at[u_idx.at[sl]], ue_b.at[s], sem),
                pltpu.async_copy(ie2.at[i_idx.at[sl]], ie_b.at[s], sem)]

    pend = fire_big(0)
    for dsc in small:
        dsc.wait()

    for c in range(nch):
        for dsc in pend:
            dsc.wait()
        if c + 1 < nch:
            pend = fire_big(c + 1)
        s = c % 2
        uev, iev = ue_b.at[s], ie_b.at[s]
        off = c * _CH

        def group(g, _):
            rows = g * _L + lax.iota(jnp.int32, _L)
            zero = jnp.zeros((_L,), jnp.float32)
            accp = [zero, zero, zero, zero]
            accn = [zero, zero, zero, zero]
            for d in range(_D):
                cols = jnp.full((_L,), d, jnp.int32)
                accp[d % 4] = accp[d % 4] + (plsc.load_gather(uev, [rows, cols])
                                             * plsc.load_gather(iev, [rows, cols]))
                colsn = jnp.full((_L,), _D + d, jnp.int32)
                accn[d % 4] = accn[d % 4] + (plsc.load_gather(uev, [rows, colsn])
                                             * plsc.load_gather(iev, [rows, colsn]))
            dot_p = (accp[0] + accp[1]) + (accp[2] + accp[3])
            dot_n = (accn[0] + accn[1]) + (accn[2] + accn[3])
            sl16 = pl.ds(off + g * _L, _L)
            pos_v[sl16] = dot_p + up_v[sl16] + ibp_v[sl16]
            neg_v[sl16] = dot_n + un_v[sl16] + ibn_v[sl16]
            return 0

        lax.fori_loop(0, _CH // _L, group, 0)

    osl = pl.ds(base, bpw)
    pltpu.sync_copy(pos_v, pos_out.at[osl])
    pltpu.sync_copy(neg_v, neg_out.at[osl])
    pltpu.sync_copy(g_v, gam_out.at[osl])
    pltpu.sync_copy(d_v, dlt_out.at[osl])
    pltpu.sync_copy(rp_v, rpo_out.at[osl])
    pltpu.sync_copy(dv0, w0_out.at[osl])
    pltpu.sync_copy(dv1, w1_out.at[osl])
    pltpu.sync_copy(dv2, w2_out.at[osl])
    pltpu.sync_copy(dv3, w3_out.at[osl])
    pltpu.sync_copy(dv4, w4_out.at[osl])


def _tc_body(gbg, gbd, gbp, gbn, pos, neg, gam, dlt, rp,
             q0, q1, q2, q3, q4, out):
    gamma = gbg[0, 0] + gam[...]
    delta = gbd[0, 0] + dlt[...]
    pv = gbp[0, 0] + pos[...]
    nv = gbn[0, 0] + neg[...]
    r = rp[...]
    acc = jnp.zeros_like(r)
    for k, q in enumerate((q0, q1, q2, q3, q4)):
        t = jnp.tanh((k + 1.0) - r)
        val = jnp.where(t > 0.0, pv * t, nv * t)
        dk = q[...]
        num = delta * jnp.exp(gamma * jnp.log(dk))
        den = num + jnp.exp(gamma * jnp.log(1.0 - dk))
        acc = acc + (num / den) * val
    out[...] = acc


def kernel(users, items, distribution, item_price, ref_point, gb_g, ub_g,
           gb_d, ub_d, gb_p, ub_p, ib_p, ue_p, ie_p, gb_n, ub_n, ib_n,
           ue_n, ie_n):
    del item_price
    B = users.shape[0]
    bpw = B // _NW
    NU = ue_p.shape[0]
    NI = ie_p.shape[0]
    f32 = jnp.float32
    u = users.astype(jnp.int32)
    it = items.astype(jnp.int32)

    grid = (NU + _BS - 1) // _BS
    tvec = jax.ShapeDtypeStruct((NI,), f32)
    ue2, ie2, d0, d1, d2, d3, d4 = pl.pallas_call(
        _prep_body,
        grid=(grid,),
        in_specs=[
            pl.BlockSpec((_D, _BS), lambda j: (0, j)),
            pl.BlockSpec((_D, _BS), lambda j: (0, j)),
            pl.BlockSpec((_D, _BS), lambda j: (0, j)),
            pl.BlockSpec((_D, _BS), lambda j: (0, j)),
            pl.BlockSpec((5, _BS), lambda j: (0, j)),
        ],
        out_specs=[
            pl.BlockSpec((_BS, 2 * _D), lambda j: (j, 0)),
            pl.BlockSpec((_BS, 2 * _D), lambda j: (j, 0)),
            pl.BlockSpec((_BS,), lambda j: (j,)),
            pl.BlockSpec((_BS,), lambda j: (j,)),
            pl.BlockSpec((_BS,), lambda j: (j,)),
            pl.BlockSpec((_BS,), lambda j: (j,)),
            pl.BlockSpec((_BS,), lambda j: (j,)),
        ],
        out_shape=[
            jax.ShapeDtypeStruct((NU, 2 * _D), f32),
            jax.ShapeDtypeStruct((NI, 2 * _D), f32),
            tvec, tvec, tvec, tvec, tvec,
        ],
    )(ue_p.T, ue_n.T, ie_p.T, ie_n.T, distribution.T)

    mesh = plsc.VectorSubcoreMesh(core_axis_name="c", subcore_axis_name="s")
    vecs = jax.ShapeDtypeStruct((B,), f32)
    sc = pl.kernel(
        _sc_body,
        out_type=[vecs] * 10,
        mesh=mesh,
        compiler_params=pltpu.CompilerParams(needs_layout_passes=False,
                                             use_tc_tiling_on_sc=True),
        scratch_types=[
            pltpu.VMEM((bpw,), jnp.int32),
            pltpu.VMEM((bpw,), jnp.int32),
            pltpu.VMEM((bpw,), f32),
            pltpu.VMEM((bpw,), f32),
            pltpu.VMEM((bpw,), f32),
            pltpu.VMEM((bpw,), f32),
            pltpu.VMEM((bpw,), f32),
            pltpu.VMEM((bpw,), f32),
            pltpu.VMEM((bpw,), f32),
            pltpu.VMEM((bpw,), f32),
            pltpu.VMEM((bpw,), f32),
            pltpu.VMEM((bpw,), f32),
            pltpu.VMEM((bpw,), f32),
            pltpu.VMEM((bpw,), f32),
            pltpu.VMEM((bpw,), f32),
            pltpu.VMEM((bpw,), f32),
            pltpu.VMEM((2, _CH, 2 * _D), f32),
            pltpu.VMEM((2, _CH, 2 * _D), f32),
            pltpu.SemaphoreType.DMA,
            pltpu.SemaphoreType.DMA,
            pltpu.SemaphoreType.DMA,
        ],
    )
    pos, neg, gam, dlt, rpo, w0, w1, w2, w3, w4 = sc(
        u, it, ue2, ie2, ref_point.reshape(-1), ub_g.reshape(-1),
        ub_d.reshape(-1), ub_p.reshape(-1), ub_n.reshape(-1),
        ib_p.reshape(-1), ib_n.reshape(-1), d0, d1, d2, d3, d4)

    M = B // 128
    r2 = lambda x: x.reshape(M, 128)
    smem = pl.BlockSpec(memory_space=pltpu.SMEM)
    vmem = pl.BlockSpec(memory_space=pltpu.VMEM)
    out2d = pl.pallas_call(
        _tc_body,
        out_shape=jax.ShapeDtypeStruct((M, 128), f32),
        in_specs=[smem] * 4 + [vmem] * 10,
        out_specs=vmem,
    )(gb_g, gb_d, gb_p, gb_n, r2(pos), r2(neg), r2(gam), r2(dlt), r2(rpo),
      r2(w0), r2(w1), r2(w2), r2(w3), r2(w4))
    return out2d.reshape(B)


# speedup vs baseline: 2.9994x; 1.5870x over previous
import functools

import jax
import jax.numpy as jnp
from jax import lax
from jax.experimental import pallas as pl
from jax.experimental.pallas import tpu as pltpu
from jax.experimental.pallas import tpu_sc as plsc

_NC = 2
_NS = 16
_NW = _NC * _NS
_L = 16
_D = 64
_CH = 128
_BS = 2048


def _bf16_hi(x):
    u = lax.bitcast_convert_type(x, jnp.uint32)
    r = u + jnp.uint32(0x7FFF) + ((u >> jnp.uint32(16)) & jnp.uint32(1))
    return r & jnp.uint32(0xFFFF0000)


def _prep_body(uepta, uepta2, uenta, uenta2, iepta, iepta2, ienta, ienta2,
               distt, ue2, ie2, d0, d1, d2, d3, d4):
    def pack(lo_t, hi_t):
        word = (_bf16_hi(lo_t.T) >> jnp.uint32(16)) | _bf16_hi(hi_t.T)
        return lax.bitcast_convert_type(word, jnp.int32)

    ue2[:, 0:_D] = pack(uepta[...], uenta[...])
    ue2[:, _D:2 * _D] = pack(uepta2[...], uenta2[...])
    ie2[:, 0:_D] = pack(iepta[...], ienta[...])
    ie2[:, _D:2 * _D] = pack(iepta2[...], ienta2[...])
    dd = distt[...]
    d0[...] = dd[0]
    d1[...] = dd[1]
    d2[...] = dd[2]
    d3[...] = dd[3]
    d4[...] = dd[4]


def _sca_body(users, items, rp_tab, ug_tab, ud_tab, up_tab, un_tab,
              ibp_tab, ibn_tab,
              rpo_out, gam_out, dlt_out, upg_out, ung_out, ibpg_out, ibng_out,
              u_idx, i_idx, rp_v, g_v, d_v, up_v, un_v, ibp_v, ibn_v, sem):
    bpw = rp_v.shape[0]
    wid = lax.axis_index("s") * _NC + lax.axis_index("c")
    base = wid * bpw
    pltpu.sync_copy(users.at[pl.ds(base, bpw)], u_idx)
    pltpu.sync_copy(items.at[pl.ds(base, bpw)], i_idx)
    pend = [
        pltpu.async_copy(rp_tab.at[u_idx], rp_v, sem),
        pltpu.async_copy(ug_tab.at[u_idx], g_v, sem),
        pltpu.async_copy(ud_tab.at[u_idx], d_v, sem),
        pltpu.async_copy(up_tab.at[u_idx], up_v, sem),
        pltpu.async_copy(un_tab.at[u_idx], un_v, sem),
        pltpu.async_copy(ibp_tab.at[i_idx], ibp_v, sem),
        pltpu.async_copy(ibn_tab.at[i_idx], ibn_v, sem),
    ]
    for dsc in pend:
        dsc.wait()
    osl = pl.ds(base, bpw)
    pltpu.sync_copy(rp_v, rpo_out.at[osl])
    pltpu.sync_copy(g_v, gam_out.at[osl])
    pltpu.sync_copy(d_v, dlt_out.at[osl])
    pltpu.sync_copy(up_v, upg_out.at[osl])
    pltpu.sync_copy(un_v, ung_out.at[osl])
    pltpu.sync_copy(ibp_v, ibpg_out.at[osl])
    pltpu.sync_copy(ibn_v, ibng_out.at[osl])


def _scb_body(users, items, ue2, ie2, d0_tab, d1_tab, d2_tab, d3_tab, d4_tab,
              pos_out, neg_out, w0_out, w1_out, w2_out, w3_out, w4_out,
              u_idx, i_idx, u_row, i_row, uo_v, io_v, dv0, dv1, dv2, dv3, dv4,
              pos_v, neg_v, ue_b, ie_b, sem_small, sem_a, sem_b,
              nu2, ni2):
    bpw = pos_v.shape[0]
    nch = bpw // _CH
    wid = lax.axis_index("s") * _NC + lax.axis_index("c")
    base = wid * bpw

    pltpu.sync_copy(users.at[pl.ds(base, bpw)], u_idx)
    pltpu.sync_copy(items.at[pl.ds(base, bpw)], i_idx)

    small = [
        pltpu.async_copy(d0_tab.at[i_idx], dv0, sem_small),
        pltpu.async_copy(d1_tab.at[i_idx], dv1, sem_small),
        pltpu.async_copy(d2_tab.at[i_idx], dv2, sem_small),
        pltpu.async_copy(d3_tab.at[i_idx], dv3, sem_small),
        pltpu.async_copy(d4_tab.at[i_idx], dv4, sem_small),
    ]

    def fix(g, _):
        sl = pl.ds(g * _L, _L)
        v = u_idx[sl]
        hi = v >= nu2
        u_row[sl] = jnp.where(hi, v - nu2, v)
        uo_v[sl] = jnp.where(hi, _D, 0)
        w = i_idx[sl]
        hj = w >= ni2
        i_row[sl] = jnp.where(hj, w - ni2, w)
        io_v[sl] = jnp.where(hj, _D, 0)
        return 0

    lax.fori_loop(0, bpw // _L, fix, 0)

    def fire_big(c):
        s = c % 2
        sem = sem_a if s == 0 else sem_b
        sl = pl.ds(c * _CH, _CH)
        return [pltpu.async_copy(ue2.at[u_row.at[sl]], ue_b.at[s], sem),
                pltpu.async_copy(ie2.at[i_row.at[sl]], ie_b.at[s], sem)]

    pend = fire_big(0)

    for c in range(nch):
        for dsc in pend:
            dsc.wait()
        if c + 1 < nch:
            pend = fire_big(c + 1)
        s = c % 2
        uev, iev = ue_b.at[s], ie_b.at[s]
        off = c * _CH

        def group(g, _):
            rows = g * _L + lax.iota(jnp.int32, _L)
            sl16 = pl.ds(off + g * _L, _L)
            ucol0 = uo_v[sl16]
            icol0 = io_v[sl16]
            zero = jnp.zeros((_L,), jnp.float32)
            accp = [zero, zero]
            accn = [zero, zero]
            for d in range(_D):
                wu = plsc.bitcast(plsc.load_gather(uev, [rows, ucol0 + d]),
                                  jnp.bfloat16)
                wi = plsc.bitcast(plsc.load_gather(iev, [rows, icol0 + d]),
                                  jnp.bfloat16)
                up, un = plsc.unpack(wu, format=plsc.PackFormat.INTERLEAVED)
                ip, in_ = plsc.unpack(wi, format=plsc.PackFormat.INTERLEAVED)
                accp[d % 2] = accp[d % 2] + up * ip
                accn[d % 2] = accn[d % 2] + un * in_
            pos_v[sl16] = accp[0] + accp[1]
            neg_v[sl16] = accn[0] + accn[1]
            return 0

        lax.fori_loop(0, _CH // _L, group, 0)

    for dsc in small:
        dsc.wait()
    osl = pl.ds(base, bpw)
    pltpu.sync_copy(pos_v, pos_out.at[osl])
    pltpu.sync_copy(neg_v, neg_out.at[osl])
    pltpu.sync_copy(dv0, w0_out.at[osl])
    pltpu.sync_copy(dv1, w1_out.at[osl])
    pltpu.sync_copy(dv2, w2_out.at[osl])
    pltpu.sync_copy(dv3, w3_out.at[osl])
    pltpu.sync_copy(dv4, w4_out.at[osl])


def _tc_body(gbg, gbd, gbp, gbn, pos, neg, upg, ung, ibpg, ibng,
             gam, dlt, rp, q0, q1, q2, q3, q4, out):
    gamma = gbg[0, 0] + gam[...]
    delta = gbd[0, 0] + dlt[...]
    pv = gbp[0, 0] + pos[...] + upg[...] + ibpg[...]
    nv = gbn[0, 0] + neg[...] + ung[...] + ibng[...]
    r = rp[...]
    acc = jnp.zeros_like(r)
    for k, q in enumerate((q0, q1, q2, q3, q4)):
        t = jnp.tanh((k + 1.0) - r)
        val = jnp.where(t > 0.0, pv * t, nv * t)
        dk = q[...]
        num = delta * jnp.exp(gamma * jnp.log(dk))
        den = num + jnp.exp(gamma * jnp.log(1.0 - dk))
        acc = acc + (num / den) * val
    out[...] = acc


def kernel(users, items, distribution, item_price, ref_point, gb_g, ub_g,
           gb_d, ub_d, gb_p, ub_p, ib_p, ue_p, ie_p, gb_n, ub_n, ib_n,
           ue_n, ie_n):
    del item_price
    B = users.shape[0]
    bpw = B // _NW
    NU = ue_p.shape[0]
    NI = ie_p.shape[0]
    nblk = -(-NU // (2 * _BS))
    hb = nblk * _BS
    nu2 = ni2 = hb
    f32 = jnp.float32
    i32 = jnp.int32
    u = users.astype(i32)
    it = items.astype(i32)
    mesh = plsc.VectorSubcoreMesh(core_axis_name="c", subcore_axis_name="s")
    vecs = jax.ShapeDtypeStruct((B,), f32)
    scp = pltpu.CompilerParams(needs_layout_passes=False,
                               use_tc_tiling_on_sc=True)

    sca = pl.kernel(
        _sca_body,
        out_type=[vecs] * 7,
        mesh=mesh,
        compiler_params=scp,
        scratch_types=[pltpu.VMEM((bpw,), i32)] * 2
        + [pltpu.VMEM((bpw,), f32)] * 7
        + [pltpu.SemaphoreType.DMA],
    )
    rpo, gam, dlt, upg, ung, ibpg, ibng = sca(
        u, it, ref_point.reshape(-1), ub_g.reshape(-1), ub_d.reshape(-1),
        ub_p.reshape(-1), ub_n.reshape(-1), ib_p.reshape(-1),
        ib_n.reshape(-1))

    grid = nblk
    half = nblk
    tspec = pl.BlockSpec((_D, _BS), lambda j: (0, j))
    last = (NU - 1) // _BS
    tspec2 = pl.BlockSpec((_D, _BS),
                          lambda j: (0, jnp.minimum(j + half, last)))
    tvec = jax.ShapeDtypeStruct((NI,), f32)
    dspec = pl.BlockSpec((2 * _BS,), lambda j: (j,))
    ue2, ie2, d0, d1, d2, d3, d4 = pl.pallas_call(
        _prep_body,
        grid=(grid,),
        in_specs=[tspec, tspec2, tspec, tspec2, tspec, tspec2, tspec, tspec2,
                  pl.BlockSpec((5, 2 * _BS), lambda j: (0, j))],
        out_specs=[
            pl.BlockSpec((_BS, 2 * _D), lambda j: (j, 0)),
            pl.BlockSpec((_BS, 2 * _D), lambda j: (j, 0)),
            dspec, dspec, dspec, dspec, dspec,
        ],
        out_shape=[
            jax.ShapeDtypeStruct((nu2, 2 * _D), i32),
            jax.ShapeDtypeStruct((ni2, 2 * _D), i32),
            tvec, tvec, tvec, tvec, tvec,
        ],
    )(ue_p.T, ue_p.T, ue_n.T, ue_n.T, ie_p.T, ie_p.T, ie_n.T, ie_n.T,
      distribution.T)

    scb = pl.kernel(
        functools.partial(_scb_body, nu2=nu2, ni2=ni2),
        out_type=[vecs] * 7,
        mesh=mesh,
        compiler_params=scp,
        scratch_types=[pltpu.VMEM((bpw,), i32)] * 6
        + [pltpu.VMEM((bpw,), f32)] * 7
        + [pltpu.VMEM((2, _CH, 2 * _D), i32)] * 2
        + [pltpu.SemaphoreType.DMA] * 3,
    )
    pos, neg, w0, w1, w2, w3, w4 = scb(u, it, ue2, ie2, d0, d1, d2, d3, d4)

    M = B // 128
    r2 = lambda x: x.reshape(M, 128)
    smem = pl.BlockSpec(memory_space=pltpu.SMEM)
    vmem = pl.BlockSpec(memory_space=pltpu.VMEM)
    out2d = pl.pallas_call(
        _tc_body,
        out_shape=jax.ShapeDtypeStruct((M, 128), f32),
        in_specs=[smem] * 4 + [vmem] * 14,
        out_specs=vmem,
    )(gb_g, gb_d, gb_p, gb_n, r2(pos), r2(neg), r2(upg), r2(ung), r2(ibpg),
      r2(ibng), r2(gam), r2(dlt), r2(rpo), r2(w0), r2(w1), r2(w2), r2(w3),
      r2(w4))
    return out2d.reshape(B)
